# Initial kernel scaffold; baseline (speedup 1.0000x reference)
import jax
import jax.numpy as jnp
from jax.experimental import pallas as pl


def kernel(y, rs):
    raise NotImplementedError("write your pallas kernel here")


# speedup vs baseline: 1.3788x; 1.3788x over previous
import jax
import jax.numpy as jnp
from jax import lax
from jax.experimental import pallas as pl
from jax.experimental.pallas import tpu as pltpu
from jax.experimental.pallas import tpu_sc as plsc

NB, NV, NF = 16, 10000, 128
NC, NS = 2, 16
NW = NC * NS
RPW = NB * NV // NW
CHUNK = 128
NCHUNK = (RPW + CHUNK - 1) // CHUNK
LAST_START = RPW - CHUNK


def _gather_body(y_hbm, rs_hbm, out_hbm, idx_v, buf_v, gsem):
    cid = lax.axis_index("c")
    sid = lax.axis_index("s")
    wid = sid * NC + cid
    b = wid // 2
    h = wid % 2
    ibase = h * RPW
    obase = wid * RPW
    boff = b * NV

    def chunk(c, carry):
        start = pl.multiple_of(jnp.minimum(c * CHUNK, LAST_START), 8)
        pltpu.sync_copy(rs_hbm.at[pl.ds(ibase + start, CHUNK)], idx_v)
        for k in range(CHUNK // 16):
            sl = pl.ds(k * 16, 16)
            idx_v[sl] = idx_v[sl] + boff
        pltpu.async_copy(y_hbm.at[idx_v], buf_v, gsem).wait()
        pltpu.sync_copy(buf_v, out_hbm.at[pl.ds(obase + start, CHUNK)])
        return carry

    lax.fori_loop(0, NCHUNK, chunk, 0)


def kernel(y, rs):
    y_flat = y.reshape(NB * NV, NF)
    rs = rs.astype(jnp.int32)
    mesh = plsc.VectorSubcoreMesh(core_axis_name="c", subcore_axis_name="s")
    out_flat = pl.kernel(
        _gather_body,
        mesh=mesh,
        out_type=jax.ShapeDtypeStruct((NB * NV, NF), jnp.float32),
        scratch_types=[
            pltpu.VMEM((CHUNK,), jnp.int32),
            pltpu.VMEM((CHUNK, NF), jnp.float32),
            pltpu.SemaphoreType.DMA,
        ],
    )(y_flat, rs)
    return out_flat.reshape(NB, NV, NF)


# speedup vs baseline: 2.2854x; 1.6575x over previous
import jax
import jax.numpy as jnp
from jax import lax
from jax.experimental import pallas as pl
from jax.experimental.pallas import tpu as pltpu
from jax.experimental.pallas import tpu_sc as plsc

NB, NV, NF = 16, 10000, 128
NC, NS = 2, 16
NW = NC * NS
RPW = NB * NV // NW
CHUNK = 128
NCHUNK = (RPW + CHUNK - 1) // CHUNK
LAST_START = RPW - CHUNK
NIDX = 5008
NBUF = 4


def _gather_body(y_hbm, rs_hbm, out_hbm, idx_v, bufs, gsems, wsems):
    cid = lax.axis_index("c")
    sid = lax.axis_index("s")
    wid = sid * NC + cid
    b = wid // 2
    h = wid % 2
    ibase = h * RPW
    obase = wid * RPW
    boff = b * NV

    pltpu.sync_copy(rs_hbm.at[pl.ds(ibase, RPW)], idx_v.at[pl.ds(0, RPW)])

    def addk(k, carry):
        sl = pl.ds(pl.multiple_of(k * 16, 16), 16)
        idx_v[sl] = idx_v[sl] + boff
        return carry

    lax.fori_loop(0, NIDX // 16, addk, 0)

    handles = {}
    for c in range(NCHUNK + 1):
        s = c % NBUF
        if c >= NBUF:
            handles[("w", c - NBUF)].wait()
        if c < NCHUNK:
            start = min(c * CHUNK, LAST_START)
            handles[("g", c)] = pltpu.async_copy(
                y_hbm.at[idx_v.at[pl.ds(start, CHUNK)]], bufs[s], gsems[s])
        if c >= 1:
            p = c - 1
            sp = p % NBUF
            startp = min(p * CHUNK, LAST_START)
            handles[("g", p)].wait()
            handles[("w", p)] = pltpu.async_copy(
                bufs[sp], out_hbm.at[pl.ds(obase + startp, CHUNK)], wsems[sp])
    for p in range(NCHUNK - NBUF + 1, NCHUNK):
        handles[("w", p)].wait()


def kernel(y, rs):
    y_flat = y.reshape(NB * NV, NF)
    rs = rs.astype(jnp.int32)
    mesh = plsc.VectorSubcoreMesh(core_axis_name="c", subcore_axis_name="s")
    out_flat = pl.kernel(
        _gather_body,
        mesh=mesh,
        out_type=jax.ShapeDtypeStruct((NB * NV, NF), jnp.float32),
        scratch_types=[
            pltpu.VMEM((NIDX,), jnp.int32),
            [pltpu.VMEM((CHUNK, NF), jnp.float32) for _ in range(NBUF)],
            [pltpu.SemaphoreType.DMA for _ in range(NBUF)],
            [pltpu.SemaphoreType.DMA for _ in range(NBUF)],
        ],
    )(y_flat, rs)
    return out_flat.reshape(NB, NV, NF)


# speedup vs baseline: 2.3134x; 1.0123x over previous
import jax
import jax.numpy as jnp
from jax import lax
from jax.experimental import pallas as pl
from jax.experimental.pallas import tpu as pltpu
from jax.experimental.pallas import tpu_sc as plsc

NB, NV, NF = 16, 10000, 128
NC, NS = 2, 16
NW = NC * NS
RPW = NB * NV // NW
CHUNK = 128
NCHUNK = (RPW + CHUNK - 1) // CHUNK
LAST_START = RPW - CHUNK
NIDX = 5008
NBUF = 6
DEPTH = 4


def _gather_body(y_hbm, rs_hbm, out_hbm, idx_v, bufs, gsems, wsems):
    cid = lax.axis_index("c")
    sid = lax.axis_index("s")
    wid = sid * NC + cid
    b = wid // 2
    h = wid % 2
    ibase = h * RPW
    obase = wid * RPW
    boff = b * NV

    pltpu.sync_copy(rs_hbm.at[pl.ds(ibase, RPW)], idx_v.at[pl.ds(0, RPW)])

    def addk(k, carry):
        sl = pl.ds(pl.multiple_of(k * 16, 16), 16)
        idx_v[sl] = idx_v[sl] + boff
        return carry

    lax.fori_loop(0, NIDX // 16, addk, 0)

    def gstart(c):
        start = min(c * CHUNK, LAST_START)
        return pltpu.async_copy(
            y_hbm.at[idx_v.at[pl.ds(start, CHUNK)]], bufs[c % NBUF],
            gsems[c % NBUF])

    handles = {}
    for c in range(DEPTH):
        handles[("g", c)] = gstart(c)
    for c in range(NCHUNK):
        s = c % NBUF
        handles[("g", c)].wait()
        handles[("w", c)] = pltpu.async_copy(
            bufs[s], out_hbm.at[pl.ds(obase + min(c * CHUNK, LAST_START),
                                      CHUNK)], wsems[s])
        n = c + DEPTH
        if n < NCHUNK:
            if n - NBUF >= 0:
                handles[("w", n - NBUF)].wait()
            handles[("g", n)] = gstart(n)
    for p in range(NCHUNK - NBUF, NCHUNK):
        handles[("w", p)].wait()


def kernel(y, rs):
    y_flat = y.reshape(NB * NV, NF)
    rs = rs.astype(jnp.int32)
    mesh = plsc.VectorSubcoreMesh(core_axis_name="c", subcore_axis_name="s")
    out_flat = pl.kernel(
        _gather_body,
        mesh=mesh,
        out_type=jax.ShapeDtypeStruct((NB * NV, NF), jnp.float32),
        scratch_types=[
            pltpu.VMEM((NIDX,), jnp.int32),
            [pltpu.VMEM((CHUNK, NF), jnp.float32) for _ in range(NBUF)],
            [pltpu.SemaphoreType.DMA for _ in range(NBUF)],
            [pltpu.SemaphoreType.DMA for _ in range(NBUF)],
        ],
    )(y_flat, rs)
    return out_flat.reshape(NB, NV, NF)


# speedup vs baseline: 2.3550x; 1.0180x over previous
import jax
import jax.numpy as jnp
from jax import lax
from jax.experimental import pallas as pl
from jax.experimental.pallas import tpu as pltpu
from jax.experimental.pallas import tpu_sc as plsc

NB, NV, NF = 16, 10000, 128
NC, NS = 2, 16
NW = NC * NS
RPW = NB * NV // NW
CHUNK = 128
CHUNKS = [(c * CHUNK, CHUNK) for c in range(RPW // CHUNK)] + [
    (RPW - RPW % CHUNK, RPW % CHUNK)]
NCHUNK = len(CHUNKS)
NIDX = 5008
NBUF = 6
DEPTH = 4
HEAD = DEPTH * CHUNK


def _gather_body(y_hbm, rs_hbm, out_hbm, idx_v, bufs, gsems, wsems):
    cid = lax.axis_index("c")
    sid = lax.axis_index("s")
    wid = sid * NC + cid
    b = wid // 2
    h = wid % 2
    ibase = h * RPW
    obase = wid * RPW
    boff = b * NV

    def add_offsets(k0, k1):
        def addk(k, carry):
            sl = pl.ds(pl.multiple_of(k * 16, 16), 16)
            idx_v[sl] = idx_v[sl] + boff
            return carry
        lax.fori_loop(k0, k1, addk, 0)

    def gstart(c):
        start, size = CHUNKS[c]
        return pltpu.async_copy(
            y_hbm.at[idx_v.at[pl.ds(start, size)]],
            bufs[c % NBUF].at[pl.ds(0, size)], gsems[c % NBUF])

    pltpu.sync_copy(rs_hbm.at[pl.ds(ibase, HEAD)], idx_v.at[pl.ds(0, HEAD)])
    add_offsets(0, HEAD // 16)
    handles = {}
    for c in range(DEPTH):
        handles[("g", c)] = gstart(c)

    pltpu.sync_copy(rs_hbm.at[pl.ds(ibase + HEAD, RPW - HEAD)],
                    idx_v.at[pl.ds(HEAD, RPW - HEAD)])
    add_offsets(HEAD // 16, NIDX // 16)

    for c in range(NCHUNK):
        s = c % NBUF
        start, size = CHUNKS[c]
        handles[("g", c)].wait()
        handles[("w", c)] = pltpu.async_copy(
            bufs[s].at[pl.ds(0, size)],
            out_hbm.at[pl.ds(obase + start, size)], wsems[s])
        n = c + DEPTH
        if n < NCHUNK:
            if n - NBUF >= 0:
                handles[("w", n - NBUF)].wait()
            handles[("g", n)] = gstart(n)
    for p in range(NCHUNK - NBUF, NCHUNK):
        handles[("w", p)].wait()


def kernel(y, rs):
    y_flat = y.reshape(NB * NV, NF)
    rs = rs.astype(jnp.int32)
    mesh = plsc.VectorSubcoreMesh(core_axis_name="c", subcore_axis_name="s")
    out_flat = pl.kernel(
        _gather_body,
        mesh=mesh,
        out_type=jax.ShapeDtypeStruct((NB * NV, NF), jnp.float32),
        scratch_types=[
            pltpu.VMEM((NIDX,), jnp.int32),
            [pltpu.VMEM((CHUNK, NF), jnp.float32) for _ in range(NBUF)],
            [pltpu.SemaphoreType.DMA for _ in range(NBUF)],
            [pltpu.SemaphoreType.DMA for _ in range(NBUF)],
        ],
    )(y_flat, rs)
    return out_flat.reshape(NB, NV, NF)
